# baseline (speedup 1.0000x reference)
import functools

import jax
import jax.numpy as jnp
from jax import lax
from jax.experimental import pallas as pl
from jax.experimental.pallas import tpu as pltpu
from jax.experimental.pallas import tpu_sc as plsc

B, S, H = 16, 2048, 128
N = B * S

_NC, _NS, _L = 2, 16, 16
_NW = _NC * _NS
_PER_W = N // _NW
_CH = 128
_NCHUNK = _PER_W // _CH

@functools.cache
def _build_sc_concept():
    mesh = plsc.VectorSubcoreMesh(core_axis_name="c", subcore_axis_name="s")

    @functools.partial(
        pl.kernel,
        mesh=mesh,
        out_type=jax.ShapeDtypeStruct((N, H), jnp.float32),
        scratch_types=[
            pltpu.VMEM((_CH,), jnp.int32),
            pltpu.VMEM((_CH,), jnp.int32),
            pltpu.VMEM((_CH,), jnp.int32),
            pltpu.VMEM((_CH, H), jnp.float32),
            pltpu.VMEM((_CH, H), jnp.float32),
            pltpu.VMEM((_CH, H), jnp.float32),
            pltpu.SemaphoreType.DMA,
        ],
    )
    def _sc_concept(concept_hbm, tt_hbm, proc_hbm, med_hbm, chart_hbm, out_hbm,
                    cid_v, tt_v, idx_v, r1_v, r2_v, r3_v, sem):
        wid = lax.axis_index("s") * _NC + lax.axis_index("c")

        def chunk(ci, carry):
            base = wid * _PER_W + ci * _CH
            pltpu.sync_copy(concept_hbm.at[pl.ds(base, _CH)], cid_v)
            pltpu.sync_copy(tt_hbm.at[pl.ds(base, _CH)], tt_v)
            for tcode, tab, dst in ((1, proc_hbm, r1_v),
                                    (2, med_hbm, r2_v),
                                    (3, chart_hbm, r3_v)):
                for j in range(_CH // _L):
                    sl = pl.ds(j * _L, _L)
                    idx_v[sl] = jnp.where(tt_v[sl] == tcode, cid_v[sl], 1)
                pltpu.async_copy(tab.at[idx_v], dst, sem).wait()

            def row(i, c2):
                for k in range(H // _L):
                    sl = pl.ds(k * _L, _L)
                    r1_v[i, sl] = r1_v[i, sl] + r2_v[i, sl] + r3_v[i, sl]
                return c2

            lax.fori_loop(0, _CH, row, 0)
            pltpu.sync_copy(r1_v, out_hbm.at[pl.ds(base, _CH)])
            return carry

        lax.fori_loop(0, _NCHUNK, chunk, 0)

    return _sc_concept


_TB = 512
_NB = N // _TB
_LN10000 = 9.210340371976184
_PI_2 = 1.5707963267948966


def _tc_body(cols_ref, cemb_ref, sel_ref, kadj_ref, stab_ref,
             pW_ref, pb_ref, tw_ref, tb_ref, tf_ref,
             vW1_ref, vb1_ref, vW2_ref, vb2_ref, out_ref):
    f32 = jnp.float32
    P = jnp.dot(cols_ref[...], sel_ref[...],
                precision=lax.Precision.HIGHEST)
    hot = (P[:, 0:256] == kadj_ref[...]).astype(f32)
    small_e = jnp.dot(hot, stab_ref[...])

    hi = lax.broadcasted_iota(jnp.int32, (1, H), 1)
    invden = jnp.exp(hi.astype(f32) * (-2.0 * _LN10000 / H))
    phase = jnp.where(hi % 2 == 0, 0.0, _PI_2)
    pos_e = jnp.sin(P[:, 256:384] * invden + phase)

    tmb = P[:, 384:512]
    u_row = jnp.dot(tw_ref[...], pW_ref[0:H, :])
    c_row = jnp.dot(tb_ref[...], pW_ref[0:H, :]) + pb_ref[...]
    per = jnp.sin(tmb * tf_ref[...] + tb_ref[...])
    time_e = tmb * u_row + c_row + jnp.dot(per, pW_ref[H:2 * H, :])

    vb = P[:, 512:640]
    h1 = jnp.maximum(vb * vW1_ref[...] + vb1_ref[...], 0.0)
    val_e = jnp.dot(h1, vW2_ref[...]) + vb2_ref[...]

    out_ref[...] = cemb_ref[...] + pos_e + small_e + time_e + val_e


def _full_spec(r, c):
    return pl.BlockSpec((r, c), lambda i: (0, 0))


_tc_call = pl.pallas_call(
    _tc_body,
    grid=(_NB,),
    in_specs=[
        pl.BlockSpec((_TB, 8), lambda i: (i, 0)),
        pl.BlockSpec((_TB, H), lambda i: (i, 0)),
        _full_spec(8, 640),
        _full_spec(1, 256),
        _full_spec(256, H),
        _full_spec(2 * H, H),
        _full_spec(1, H),
        _full_spec(1, H),
        _full_spec(1, H),
        _full_spec(1, H),
        _full_spec(1, H),
        _full_spec(1, H),
        _full_spec(H, H),
        _full_spec(1, H),
    ],
    out_specs=pl.BlockSpec((_TB, H), lambda i: (i, 0)),
    out_shape=jax.ShapeDtypeStruct((N, H), jnp.float32),
)


def _selector_constants():
    k = jnp.arange(640)
    sel = jnp.zeros((8, 640), jnp.float32)
    sel = sel.at[0].set(jnp.where(k < 128, 1.0, 0.0))
    sel = sel.at[1].set(jnp.where((k >= 128) & (k < 192), 1.0, 0.0))
    sel = sel.at[2].set(jnp.where((k >= 192) & (k < 195), 1.0, 0.0))
    sel = sel.at[3].set(jnp.where((k >= 195) & (k < 203), 1.0, 0.0))
    sel = sel.at[4].set(jnp.where((k >= 256) & (k < 384), 1.0, 0.0))
    sel = sel.at[5].set(jnp.where((k >= 384) & (k < 512), 1.0, 0.0))
    sel = sel.at[6].set(jnp.where((k >= 512) & (k < 640), 1.0, 0.0))
    k256 = jnp.arange(256)
    kadj = jnp.where(k256 < 128, k256.astype(jnp.float32), -1.0)
    kadj = jnp.where((k256 >= 128) & (k256 < 192), (k256 - 128).astype(jnp.float32), kadj)
    kadj = jnp.where((k256 >= 192) & (k256 < 195), (k256 - 192).astype(jnp.float32), kadj)
    kadj = jnp.where((k256 >= 195) & (k256 < 203), (k256 - 195).astype(jnp.float32), kadj)
    return sel, kadj.reshape(1, 256)


def kernel(concept, token_type, age, position, time, value, unit, gender, task,
           proc_table, med_table, chart_table, age_table, unit_table,
           gender_table, task_table, tw, tb, tfreqs, proj_W, proj_b,
           vW1, vb1, vW2, vb2):
    cemb = _build_sc_concept()(concept.reshape(N).astype(jnp.int32),
                               token_type.reshape(N).astype(jnp.int32),
                               proc_table, med_table, chart_table)

    stab = jnp.zeros((256, H), jnp.float32)
    stab = (stab.at[0:120].set(age_table)
                .at[128:192].set(unit_table)
                .at[192:195].set(gender_table)
                .at[195:203].set(task_table))

    f32 = jnp.float32
    cols = jnp.stack(
        [age.astype(f32), unit.astype(f32), gender.astype(f32),
         task.astype(f32), position.astype(f32), time, value,
         jnp.zeros_like(time)], axis=-1).reshape(N, 8)
    sel, kadj = _selector_constants()

    out = _tc_call(cols, cemb, sel, kadj, stab, proj_W,
                   proj_b.reshape(1, H), tw, tb.reshape(1, H),
                   tfreqs.reshape(1, H), vW1, vb1.reshape(1, H),
                   vW2, vb2.reshape(1, H))
    return out.reshape(B, S, H)


# speedup vs baseline: 5.0058x; 5.0058x over previous
import functools

import jax
import jax.numpy as jnp
from jax import lax
from jax.experimental import pallas as pl
from jax.experimental.pallas import tpu as pltpu
from jax.experimental.pallas import tpu_sc as plsc

B, S, H = 16, 2048, 128
N = B * S

_NC, _NS, _L = 2, 16, 16
_NW = _NC * _NS
_PER_W = N // _NW
_CH = 128
_NCHUNK = _PER_W // _CH

@functools.cache
def _build_sc_concept():
    mesh = plsc.VectorSubcoreMesh(core_axis_name="c", subcore_axis_name="s")

    @functools.partial(
        pl.kernel,
        mesh=mesh,
        out_type=jax.ShapeDtypeStruct((N, 3 * H), jnp.float32),
        scratch_types=[
            pltpu.VMEM((_CH,), jnp.int32),
            pltpu.VMEM((_CH, H), jnp.float32),
            pltpu.VMEM((_CH, H), jnp.float32),
            pltpu.VMEM((_CH, H), jnp.float32),
            pltpu.SemaphoreType.DMA,
        ],
    )
    def _sc_concept(concept_hbm, proc_hbm, med_hbm, chart_hbm, out_hbm,
                    cid_v, r1_v, r2_v, r3_v, sem):
        wid = lax.axis_index("s") * _NC + lax.axis_index("c")

        def chunk(ci, carry):
            base = wid * _PER_W + ci * _CH
            pltpu.sync_copy(concept_hbm.at[pl.ds(base, _CH)], cid_v)
            g1 = pltpu.async_copy(proc_hbm.at[cid_v], r1_v, sem)
            g2 = pltpu.async_copy(med_hbm.at[cid_v], r2_v, sem)
            g3 = pltpu.async_copy(chart_hbm.at[cid_v], r3_v, sem)
            g1.wait()
            g2.wait()
            g3.wait()
            rows = out_hbm.at[pl.ds(base, _CH)]
            pltpu.sync_copy(r1_v, rows.at[:, pl.ds(0, H)])
            pltpu.sync_copy(r2_v, rows.at[:, pl.ds(H, H)])
            pltpu.sync_copy(r3_v, rows.at[:, pl.ds(2 * H, H)])
            return carry

        lax.fori_loop(0, _NCHUNK, chunk, 0)

    return _sc_concept


_TB = 512
_NB = N // _TB
_LN10000 = 9.210340371976184
_PI_2 = 1.5707963267948966


def _tc_body(cols_ref, cemb_ref, sel_ref, kadj_ref, stab_ref,
             pW_ref, pb_ref, tw_ref, tb_ref, tf_ref,
             vW1_ref, vb1_ref, vW2_ref, vb2_ref, out_ref):
    f32 = jnp.float32
    P = jnp.dot(cols_ref[...], sel_ref[...],
                precision=lax.Precision.HIGHEST)
    hot = (P[:, 0:256] == kadj_ref[...]).astype(f32)
    small_e = jnp.dot(hot, stab_ref[...])

    hi = lax.broadcasted_iota(jnp.int32, (1, H), 1)
    invden = jnp.exp(hi.astype(f32) * (-2.0 * _LN10000 / H))
    phase = jnp.where(hi % 2 == 0, 0.0, _PI_2)
    pos_e = jnp.sin(P[:, 256:384] * invden + phase)

    tmb = P[:, 384:512]
    u_row = jnp.dot(tw_ref[...], pW_ref[0:H, :])
    c_row = jnp.dot(tb_ref[...], pW_ref[0:H, :]) + pb_ref[...]
    per = jnp.sin(tmb * tf_ref[...] + tb_ref[...])
    time_e = tmb * u_row + c_row + jnp.dot(per, pW_ref[H:2 * H, :])

    vb = P[:, 512:640]
    h1 = jnp.maximum(vb * vW1_ref[...] + vb1_ref[...], 0.0)
    val_e = jnp.dot(h1, vW2_ref[...]) + vb2_ref[...]

    ttb = P[:, 640:768]
    zero = jnp.zeros((), f32)
    cemb = (jnp.where(ttb == 1.0, cemb_ref[:, 0:H], zero)
            + jnp.where(ttb == 2.0, cemb_ref[:, H:2 * H], zero)
            + jnp.where(ttb == 3.0, cemb_ref[:, 2 * H:3 * H], zero))

    out_ref[...] = cemb + pos_e + small_e + time_e + val_e


def _full_spec(r, c):
    return pl.BlockSpec((r, c), lambda i: (0, 0))


_tc_call = pl.pallas_call(
    _tc_body,
    grid=(_NB,),
    in_specs=[
        pl.BlockSpec((_TB, 8), lambda i: (i, 0)),
        pl.BlockSpec((_TB, 3 * H), lambda i: (i, 0)),
        _full_spec(8, 768),
        _full_spec(1, 256),
        _full_spec(256, H),
        _full_spec(2 * H, H),
        _full_spec(1, H),
        _full_spec(1, H),
        _full_spec(1, H),
        _full_spec(1, H),
        _full_spec(1, H),
        _full_spec(1, H),
        _full_spec(H, H),
        _full_spec(1, H),
    ],
    out_specs=pl.BlockSpec((_TB, H), lambda i: (i, 0)),
    out_shape=jax.ShapeDtypeStruct((N, H), jnp.float32),
)


def _selector_constants():
    k = jnp.arange(768)
    sel = jnp.zeros((8, 768), jnp.float32)
    sel = sel.at[0].set(jnp.where(k < 128, 1.0, 0.0))
    sel = sel.at[1].set(jnp.where((k >= 128) & (k < 192), 1.0, 0.0))
    sel = sel.at[2].set(jnp.where((k >= 192) & (k < 195), 1.0, 0.0))
    sel = sel.at[3].set(jnp.where((k >= 195) & (k < 203), 1.0, 0.0))
    sel = sel.at[4].set(jnp.where((k >= 256) & (k < 384), 1.0, 0.0))
    sel = sel.at[5].set(jnp.where((k >= 384) & (k < 512), 1.0, 0.0))
    sel = sel.at[6].set(jnp.where((k >= 512) & (k < 640), 1.0, 0.0))
    sel = sel.at[7].set(jnp.where(k >= 640, 1.0, 0.0))
    k256 = jnp.arange(256)
    kadj = jnp.where(k256 < 128, k256.astype(jnp.float32), -1.0)
    kadj = jnp.where((k256 >= 128) & (k256 < 192), (k256 - 128).astype(jnp.float32), kadj)
    kadj = jnp.where((k256 >= 192) & (k256 < 195), (k256 - 192).astype(jnp.float32), kadj)
    kadj = jnp.where((k256 >= 195) & (k256 < 203), (k256 - 195).astype(jnp.float32), kadj)
    return sel, kadj.reshape(1, 256)


def kernel(concept, token_type, age, position, time, value, unit, gender, task,
           proc_table, med_table, chart_table, age_table, unit_table,
           gender_table, task_table, tw, tb, tfreqs, proj_W, proj_b,
           vW1, vb1, vW2, vb2):
    cemb = _build_sc_concept()(concept.reshape(N).astype(jnp.int32),
                               proc_table, med_table, chart_table)

    stab = jnp.zeros((256, H), jnp.float32)
    stab = (stab.at[0:120].set(age_table)
                .at[128:192].set(unit_table)
                .at[192:195].set(gender_table)
                .at[195:203].set(task_table))

    f32 = jnp.float32
    cols = jnp.stack(
        [age.astype(f32), unit.astype(f32), gender.astype(f32),
         task.astype(f32), position.astype(f32), time, value,
         token_type.astype(f32)], axis=-1).reshape(N, 8)
    sel, kadj = _selector_constants()

    out = _tc_call(cols, cemb, sel, kadj, stab, proj_W,
                   proj_b.reshape(1, H), tw, tb.reshape(1, H),
                   tfreqs.reshape(1, H), vW1, vb1.reshape(1, H),
                   vW2, vb2.reshape(1, H))
    return out.reshape(B, S, H)


# speedup vs baseline: 10.7332x; 2.1442x over previous
import functools

import jax
import jax.numpy as jnp
from jax import lax
from jax.experimental import pallas as pl
from jax.experimental.pallas import tpu as pltpu
from jax.experimental.pallas import tpu_sc as plsc

B, S, H = 16, 2048, 128
N = B * S

_NC, _NS, _L = 2, 16, 16
_NW = _NC * _NS
_PER_W = N // _NW
_CH = 128
_NCHUNK = _PER_W // _CH

@functools.cache
def _build_sc_concept():
    mesh = plsc.VectorSubcoreMesh(core_axis_name="c", subcore_axis_name="s")

    @functools.partial(
        pl.kernel,
        mesh=mesh,
        out_type=jax.ShapeDtypeStruct((N, 3 * H), jnp.float32),
        scratch_types=[
            pltpu.VMEM((_CH,), jnp.int32),
            pltpu.VMEM((_CH, H), jnp.float32),
            pltpu.VMEM((_CH, H), jnp.float32),
            pltpu.VMEM((_CH, H), jnp.float32),
            pltpu.SemaphoreType.DMA,
        ],
    )
    def _sc_concept(concept_hbm, proc_hbm, med_hbm, chart_hbm, out_hbm,
                    cid_v, r1_v, r2_v, r3_v, sem):
        wid = lax.axis_index("s") * _NC + lax.axis_index("c")

        def chunk(ci, carry):
            base = wid * _PER_W + ci * _CH
            pltpu.sync_copy(concept_hbm.at[pl.ds(base, _CH)], cid_v)
            g1 = pltpu.async_copy(proc_hbm.at[cid_v], r1_v, sem)
            g2 = pltpu.async_copy(med_hbm.at[cid_v], r2_v, sem)
            g3 = pltpu.async_copy(chart_hbm.at[cid_v], r3_v, sem)
            g1.wait()
            g2.wait()
            g3.wait()
            rows = out_hbm.at[pl.ds(base, _CH)]
            pltpu.sync_copy(r1_v, rows.at[:, pl.ds(0, H)])
            pltpu.sync_copy(r2_v, rows.at[:, pl.ds(H, H)])
            pltpu.sync_copy(r3_v, rows.at[:, pl.ds(2 * H, H)])
            return carry

        lax.fori_loop(0, _NCHUNK, chunk, 0)

    return _sc_concept


_TB = 1024
_NB = N // _TB
_PI_2 = 1.5707963267948966


def _tc_body(cols_ref, cemb_ref, sel_ref, kadj_ref, stab_ref, tabs_ref,
             pW_ref, pb_ref, tw_ref, tb_ref,
             vW1_ref, vb1_ref, vW2_ref, vb2_ref, out_ref):
    f32 = jnp.float32
    P = jnp.dot(cols_ref[...], sel_ref[...])
    hot = (P[:, 0:256] == kadj_ref[...]).astype(f32)
    small_e = jnp.dot(hot, stab_ref[...])

    lane = lax.broadcasted_iota(jnp.int32, (1, H), 1).astype(f32)

    A = jnp.dot((P[:, 256:384] == lane).astype(f32), tabs_ref[:, 0:2 * H])
    Bc = jnp.dot((P[:, 384:512] == lane).astype(f32), tabs_ref[:, 2 * H:4 * H])
    pos_e = A[:, 0:H] * Bc[:, 0:H] + A[:, H:2 * H] * Bc[:, H:2 * H]

    TA = jnp.dot((P[:, 512:640] == lane).astype(f32), tabs_ref[:, 4 * H:6 * H])
    TB = jnp.dot((P[:, 640:768] == lane).astype(f32), tabs_ref[:, 6 * H:8 * H])
    per = TA[:, 0:H] * TB[:, 0:H] + TA[:, H:2 * H] * TB[:, H:2 * H]

    tmb = P[:, 768:896]
    u_row = jnp.dot(tw_ref[...], pW_ref[0:H, :])
    c_row = jnp.dot(tb_ref[...], pW_ref[0:H, :]) + pb_ref[...]
    time_e = tmb * u_row + c_row + jnp.dot(per, pW_ref[H:2 * H, :])

    vb = P[:, 896:1024]
    h1 = jnp.maximum(vb * vW1_ref[...] + vb1_ref[...], 0.0)
    val_e = jnp.dot(h1, vW2_ref[...]) + vb2_ref[...]

    ttb = P[:, 1024:1152]
    zero = jnp.zeros((), f32)
    cemb = (jnp.where(ttb == 1.0, cemb_ref[:, 0:H], zero)
            + jnp.where(ttb == 2.0, cemb_ref[:, H:2 * H], zero)
            + jnp.where(ttb == 3.0, cemb_ref[:, 2 * H:3 * H], zero))

    out_ref[...] = cemb + pos_e + small_e + time_e + val_e


def _full_spec(r, c):
    return pl.BlockSpec((r, c), lambda i: (0, 0))


_tc_call = pl.pallas_call(
    _tc_body,
    grid=(_NB,),
    in_specs=[
        pl.BlockSpec((_TB, 16), lambda i: (i, 0)),
        pl.BlockSpec((_TB, 3 * H), lambda i: (i, 0)),
        _full_spec(16, 1152),
        _full_spec(1, 256),
        _full_spec(256, H),
        _full_spec(H, 8 * H),
        _full_spec(2 * H, H),
        _full_spec(1, H),
        _full_spec(1, H),
        _full_spec(1, H),
        _full_spec(1, H),
        _full_spec(1, H),
        _full_spec(H, H),
        _full_spec(1, H),
    ],
    out_specs=pl.BlockSpec((_TB, H), lambda i: (i, 0)),
    out_shape=jax.ShapeDtypeStruct((N, H), jnp.float32),
)


def _selector_constants():
    k = jnp.arange(1152)
    sel = jnp.zeros((16, 1152), jnp.float32)
    sel = sel.at[0].set(jnp.where(k < 128, 1.0, 0.0))
    sel = sel.at[1].set(jnp.where((k >= 128) & (k < 192), 1.0, 0.0))
    sel = sel.at[2].set(jnp.where((k >= 192) & (k < 195), 1.0, 0.0))
    sel = sel.at[3].set(jnp.where((k >= 195) & (k < 203), 1.0, 0.0))
    for c in range(7):
        sel = sel.at[4 + c].set(
            jnp.where((k >= 256 + 128 * c) & (k < 384 + 128 * c), 1.0, 0.0))
    k256 = jnp.arange(256)
    kadj = jnp.where(k256 < 128, k256.astype(jnp.float32), -1.0)
    kadj = jnp.where((k256 >= 128) & (k256 < 192), (k256 - 128).astype(jnp.float32), kadj)
    kadj = jnp.where((k256 >= 192) & (k256 < 195), (k256 - 192).astype(jnp.float32), kadj)
    kadj = jnp.where((k256 >= 195) & (k256 < 203), (k256 - 195).astype(jnp.float32), kadj)
    return sel, kadj.reshape(1, 256)


def _angle_tables(tfreqs, tb):
    f32 = jnp.float32
    i = jnp.arange(H, dtype=f32)
    w = jnp.power(10000.0, -2.0 * i / H)[None, :]
    ph = jnp.where(jnp.arange(H) % 2 == 0, 0.0, _PI_2)[None, :]
    n = jnp.arange(H, dtype=f32)[:, None]
    arg_a = 64.0 * n * w + ph
    arg_b = n * w
    f = tfreqs[None, :]
    arg_ta = (n / 64.0) * f + tb[None, :]
    arg_tb = (n / 4096.0) * f
    return jnp.concatenate(
        [jnp.sin(arg_a), jnp.cos(arg_a), jnp.cos(arg_b), jnp.sin(arg_b),
         jnp.sin(arg_ta), jnp.cos(arg_ta), jnp.cos(arg_tb), jnp.sin(arg_tb)],
        axis=1)


def kernel(concept, token_type, age, position, time, value, unit, gender, task,
           proc_table, med_table, chart_table, age_table, unit_table,
           gender_table, task_table, tw, tb, tfreqs, proj_W, proj_b,
           vW1, vb1, vW2, vb2):
    cemb = _build_sc_concept()(concept.reshape(N).astype(jnp.int32),
                               proc_table, med_table, chart_table)

    stab = jnp.zeros((256, H), jnp.float32)
    stab = (stab.at[0:120].set(age_table)
                .at[128:192].set(unit_table)
                .at[192:195].set(gender_table)
                .at[195:203].set(task_table))

    f32 = jnp.float32
    tq = jnp.minimum(jnp.floor(time * 4096.0), 4095.0)
    t_hi = jnp.floor(tq * (1.0 / 64.0))
    t_lo = tq - 64.0 * t_hi
    z = jnp.zeros_like(time)
    cols = jnp.stack(
        [age.astype(f32), unit.astype(f32), gender.astype(f32),
         task.astype(f32), (position // 64).astype(f32),
         (position % 64).astype(f32), t_hi, t_lo, time, value,
         token_type.astype(f32), z, z, z, z, z], axis=-1).reshape(N, 16)
    sel, kadj = _selector_constants()
    tabs = _angle_tables(tfreqs, tb)

    out = _tc_call(cols, cemb, sel, kadj, stab, tabs, proj_W,
                   proj_b.reshape(1, H), tw, tb.reshape(1, H),
                   vW1, vb1.reshape(1, H), vW2, vb2.reshape(1, H))
    return out.reshape(B, S, H)


# speedup vs baseline: 13.1825x; 1.2282x over previous
import functools

import jax
import jax.numpy as jnp
from jax import lax
from jax.experimental import pallas as pl
from jax.experimental.pallas import tpu as pltpu
from jax.experimental.pallas import tpu_sc as plsc

B, S, H = 16, 2048, 128
N = B * S

_NC, _NS, _L = 2, 16, 16
_NW = _NC * _NS
_PER_W = N // _NW
_CH = 128
_NCHUNK = _PER_W // _CH

_TRASH = _NW * 3 * _CH
_NBUF = 4


@functools.cache
def _build_sc_concept():
    mesh = plsc.VectorSubcoreMesh(core_axis_name="c", subcore_axis_name="s")

    @functools.partial(
        pl.kernel,
        mesh=mesh,
        out_type=jax.ShapeDtypeStruct((N + _TRASH, H), jnp.float32),
        scratch_types=[
            pltpu.VMEM((_PER_W,), jnp.int32),
            pltpu.VMEM((_PER_W,), jnp.int32),
            pltpu.VMEM((_NCHUNK, _CH), jnp.int32),
            pltpu.VMEM((_NCHUNK, _CH), jnp.int32),
            pltpu.VMEM((_NCHUNK, _CH), jnp.int32),
            pltpu.VMEM((_NCHUNK, _CH), jnp.int32),
            pltpu.VMEM((_NCHUNK, _CH), jnp.int32),
            pltpu.VMEM((_NCHUNK, _CH), jnp.int32),
            pltpu.VMEM((_NBUF, _CH, H), jnp.float32),
            pltpu.SemaphoreType.DMA,
            pltpu.SemaphoreType.DMA,
        ],
        compiler_params=pltpu.CompilerParams(needs_layout_passes=False),
    )
    def _sc_concept(concept_hbm, tt_hbm, proc_hbm, med_hbm, chart_hbm,
                    out_hbm, cid_v, tt_v, g1_v, g2_v, g3_v, s1_v, s2_v, s3_v,
                    bufs_v, gsem, ssem):
        wid = lax.axis_index("s") * _NC + lax.axis_index("c")
        base = wid * _PER_W
        pltpu.sync_copy(concept_hbm.at[pl.ds(base, _PER_W)], cid_v)
        pltpu.sync_copy(tt_hbm.at[pl.ds(base, _PER_W)], tt_v)
        i16 = jnp.arange(16, dtype=jnp.int32)

        gl = [g1_v, g2_v, g3_v]
        sl_ = [s1_v, s2_v, s3_v]
        for t in range(3):
            tbase = N + (wid * 3 + t) * _CH
            for c in range(_CH // _L):
                fill_g = i16 + (c * _L)
                fill_s = i16 + (tbase + c * _L)
                for r in range(_NCHUNK):
                    gl[t][r, pl.ds(c * _L, _L)] = fill_g
                    sl_[t][r, pl.ds(c * _L, _L)] = fill_s

        def cvec(j, ns):
            sl = pl.ds(j * _L, _L)
            c16 = cid_v[sl]
            t16 = tt_v[sl]
            rowg = (base + j * _L) + i16
            new_ns = []
            for t in range(3):
                n_t = ns[t]
                m = t16 == (t + 1)
                mi = jnp.where(m, jnp.full((_L,), 1, jnp.int32),
                               jnp.zeros((_L,), jnp.int32))
                cs = plsc.cumsum(mi)
                dst = n_t + cs - jnp.full((_L,), 1, jnp.int32)
                dhi = jnp.right_shift(dst, jnp.full((_L,), 7, jnp.int32))
                dlo = jnp.bitwise_and(dst, jnp.full((_L,), 127, jnp.int32))
                plsc.store_scatter(gl[t], [dhi, dlo], c16, mask=m)
                plsc.store_scatter(sl_[t], [dhi, dlo], rowg, mask=m)
                new_ns.append(n_t + plsc.all_reduce_population_count(m))
            return tuple(new_ns)

        zeros = jnp.zeros((_L,), jnp.int32)
        ns = (zeros, zeros, zeros)
        for j in range(_PER_W // _L):
            ns = cvec(j, ns)
        counts = [jnp.max(ns[t]) for t in range(3)]
        tabs = [proc_hbm, med_hbm, chart_hbm]

        def g_copy(i):
            t, k = i // _NCHUNK, i % _NCHUNK
            return pltpu.make_async_copy(
                tabs[t].at[gl[t].at[k]], bufs_v.at[i % _NBUF], gsem)

        def s_copy(i):
            t, k = i // _NCHUNK, i % _NCHUNK
            return pltpu.make_async_copy(
                bufs_v.at[i % _NBUF], out_hbm.at[sl_[t].at[k]], ssem)

        def active(i):
            t, k = i // _NCHUNK, i % _NCHUNK
            return k * _CH < counts[t]

        NS = 3 * _NCHUNK
        for i in range(NS + _NBUF):
            if 0 <= i - _NBUF < NS:
                pl.when(active(i - _NBUF))(lambda ii=i - _NBUF: s_copy(ii).wait())
            if i < NS:
                pl.when(active(i))(lambda ii=i: g_copy(ii).start())
            if 0 <= i - 1 < NS:
                def _fire(ii=i - 1):
                    g_copy(ii).wait()
                    s_copy(ii).start()
                pl.when(active(i - 1))(_fire)

    return _sc_concept


_TB = 1024
_NB = N // _TB
_PI_2 = 1.5707963267948966


def _tc_body(cols_ref, cemb_ref, sel_ref, kadj_ref, stab_ref, tabs_ref,
             pW_ref, pb_ref, tw_ref, tb_ref,
             vW1_ref, vb1_ref, vW2_ref, vb2_ref, out_ref):
    f32 = jnp.float32
    P = jnp.dot(cols_ref[...], sel_ref[...])
    hot = (P[:, 0:256] == kadj_ref[...]).astype(f32)
    small_e = jnp.dot(hot, stab_ref[...])

    lane = lax.broadcasted_iota(jnp.int32, (1, H), 1).astype(f32)

    A = jnp.dot((P[:, 256:384] == lane).astype(f32), tabs_ref[:, 0:2 * H])
    Bc = jnp.dot((P[:, 384:512] == lane).astype(f32), tabs_ref[:, 2 * H:4 * H])
    pos_e = A[:, 0:H] * Bc[:, 0:H] + A[:, H:2 * H] * Bc[:, H:2 * H]

    TA = jnp.dot((P[:, 512:640] == lane).astype(f32), tabs_ref[:, 4 * H:6 * H])
    TB = jnp.dot((P[:, 640:768] == lane).astype(f32), tabs_ref[:, 6 * H:8 * H])
    per = TA[:, 0:H] * TB[:, 0:H] + TA[:, H:2 * H] * TB[:, H:2 * H]

    tmb = P[:, 768:896]
    u_row = jnp.dot(tw_ref[...], pW_ref[0:H, :])
    c_row = jnp.dot(tb_ref[...], pW_ref[0:H, :]) + pb_ref[...]
    time_e = tmb * u_row + c_row + jnp.dot(per, pW_ref[H:2 * H, :])

    vb = P[:, 896:1024]
    h1 = jnp.maximum(vb * vW1_ref[...] + vb1_ref[...], 0.0)
    val_e = jnp.dot(h1, vW2_ref[...]) + vb2_ref[...]

    ttb = P[:, 1024:1152]
    cemb = jnp.where(ttb >= 1.0, cemb_ref[...], jnp.zeros((), f32))

    out_ref[...] = cemb + pos_e + small_e + time_e + val_e


def _full_spec(r, c):
    return pl.BlockSpec((r, c), lambda i: (0, 0))


_tc_call = pl.pallas_call(
    _tc_body,
    grid=(_NB,),
    in_specs=[
        pl.BlockSpec((_TB, 16), lambda i: (i, 0)),
        pl.BlockSpec((_TB, H), lambda i: (i, 0)),
        _full_spec(16, 1152),
        _full_spec(1, 256),
        _full_spec(256, H),
        _full_spec(H, 8 * H),
        _full_spec(2 * H, H),
        _full_spec(1, H),
        _full_spec(1, H),
        _full_spec(1, H),
        _full_spec(1, H),
        _full_spec(1, H),
        _full_spec(H, H),
        _full_spec(1, H),
    ],
    out_specs=pl.BlockSpec((_TB, H), lambda i: (i, 0)),
    out_shape=jax.ShapeDtypeStruct((N, H), jnp.float32),
)


def _selector_constants():
    k = jnp.arange(1152)
    sel = jnp.zeros((16, 1152), jnp.float32)
    sel = sel.at[0].set(jnp.where(k < 128, 1.0, 0.0))
    sel = sel.at[1].set(jnp.where((k >= 128) & (k < 192), 1.0, 0.0))
    sel = sel.at[2].set(jnp.where((k >= 192) & (k < 195), 1.0, 0.0))
    sel = sel.at[3].set(jnp.where((k >= 195) & (k < 203), 1.0, 0.0))
    for c in range(7):
        sel = sel.at[4 + c].set(
            jnp.where((k >= 256 + 128 * c) & (k < 384 + 128 * c), 1.0, 0.0))
    k256 = jnp.arange(256)
    kadj = jnp.where(k256 < 128, k256.astype(jnp.float32), -1.0)
    kadj = jnp.where((k256 >= 128) & (k256 < 192), (k256 - 128).astype(jnp.float32), kadj)
    kadj = jnp.where((k256 >= 192) & (k256 < 195), (k256 - 192).astype(jnp.float32), kadj)
    kadj = jnp.where((k256 >= 195) & (k256 < 203), (k256 - 195).astype(jnp.float32), kadj)
    return sel, kadj.reshape(1, 256)


def _angle_tables(tfreqs, tb):
    f32 = jnp.float32
    i = jnp.arange(H, dtype=f32)
    w = jnp.power(10000.0, -2.0 * i / H)[None, :]
    ph = jnp.where(jnp.arange(H) % 2 == 0, 0.0, _PI_2)[None, :]
    n = jnp.arange(H, dtype=f32)[:, None]
    arg_a = 64.0 * n * w + ph
    arg_b = n * w
    f = tfreqs[None, :]
    arg_ta = (n / 64.0) * f + tb[None, :]
    arg_tb = (n / 4096.0) * f
    return jnp.concatenate(
        [jnp.sin(arg_a), jnp.cos(arg_a), jnp.cos(arg_b), jnp.sin(arg_b),
         jnp.sin(arg_ta), jnp.cos(arg_ta), jnp.cos(arg_tb), jnp.sin(arg_tb)],
        axis=1)


def kernel(concept, token_type, age, position, time, value, unit, gender, task,
           proc_table, med_table, chart_table, age_table, unit_table,
           gender_table, task_table, tw, tb, tfreqs, proj_W, proj_b,
           vW1, vb1, vW2, vb2):
    cemb = _build_sc_concept()(concept.reshape(N).astype(jnp.int32),
                               token_type.reshape(N).astype(jnp.int32),
                               proc_table, med_table, chart_table)

    stab = jnp.zeros((256, H), jnp.float32)
    stab = (stab.at[0:120].set(age_table)
                .at[128:192].set(unit_table)
                .at[192:195].set(gender_table)
                .at[195:203].set(task_table))

    f32 = jnp.float32
    tq = jnp.minimum(jnp.floor(time * 4096.0), 4095.0)
    t_hi = jnp.floor(tq * (1.0 / 64.0))
    t_lo = tq - 64.0 * t_hi
    z = jnp.zeros_like(time)
    cols = jnp.stack(
        [age.astype(f32), unit.astype(f32), gender.astype(f32),
         task.astype(f32), (position // 64).astype(f32),
         (position % 64).astype(f32), t_hi, t_lo, time, value,
         token_type.astype(f32), z, z, z, z, z], axis=-1).reshape(N, 16)
    sel, kadj = _selector_constants()
    tabs = _angle_tables(tfreqs, tb)

    out = _tc_call(cols, cemb, sel, kadj, stab, tabs, proj_W,
                   proj_b.reshape(1, H), tw, tb.reshape(1, H),
                   vW1, vb1.reshape(1, H), vW2, vb2.reshape(1, H))
    return out.reshape(B, S, H)


# speedup vs baseline: 13.5732x; 1.0296x over previous
import functools

import jax
import jax.numpy as jnp
from jax import lax
from jax.experimental import pallas as pl
from jax.experimental.pallas import tpu as pltpu
from jax.experimental.pallas import tpu_sc as plsc

B, S, H = 16, 2048, 128
N = B * S

_NC, _NS, _L = 2, 16, 16
_NW = _NC * _NS
_PER_W = N // _NW
_CH = 128
_NCHUNK = _PER_W // _CH

_TRASH = _NW * 3 * _CH
_NBUF = 4


@functools.cache
def _build_sc_concept():
    mesh = plsc.VectorSubcoreMesh(core_axis_name="c", subcore_axis_name="s")

    @functools.partial(
        pl.kernel,
        mesh=mesh,
        out_type=jax.ShapeDtypeStruct((N + _TRASH, H), jnp.float32),
        scratch_types=[
            pltpu.VMEM((_PER_W,), jnp.int32),
            pltpu.VMEM((_PER_W,), jnp.int32),
            pltpu.VMEM((_NCHUNK, _CH), jnp.int32),
            pltpu.VMEM((_NCHUNK, _CH), jnp.int32),
            pltpu.VMEM((_NCHUNK, _CH), jnp.int32),
            pltpu.VMEM((_NCHUNK, _CH), jnp.int32),
            pltpu.VMEM((_NCHUNK, _CH), jnp.int32),
            pltpu.VMEM((_NCHUNK, _CH), jnp.int32),
            pltpu.VMEM((_NBUF, _CH, H), jnp.float32),
            pltpu.SemaphoreType.DMA,
            pltpu.SemaphoreType.DMA,
        ],
        compiler_params=pltpu.CompilerParams(needs_layout_passes=False),
    )
    def _sc_concept(concept_hbm, tt_hbm, proc_hbm, med_hbm, chart_hbm,
                    out_hbm, cid_v, tt_v, g1_v, g2_v, g3_v, s1_v, s2_v, s3_v,
                    bufs_v, gsem, ssem):
        wid = lax.axis_index("s") * _NC + lax.axis_index("c")
        base = wid * _PER_W
        pltpu.sync_copy(concept_hbm.at[pl.ds(base, _PER_W)], cid_v)
        pltpu.sync_copy(tt_hbm.at[pl.ds(base, _PER_W)], tt_v)
        i16 = jnp.arange(16, dtype=jnp.int32)

        gl = [g1_v, g2_v, g3_v]
        sl_ = [s1_v, s2_v, s3_v]
        for t in range(3):
            tbase = N + (wid * 3 + t) * _CH
            for c in range(_CH // _L):
                fill_g = i16 + (c * _L)
                fill_s = i16 + (tbase + c * _L)
                for r in range(_NCHUNK):
                    gl[t][r, pl.ds(c * _L, _L)] = fill_g
                    sl_[t][r, pl.ds(c * _L, _L)] = fill_s

        def cvec(j, ns):
            sl = pl.ds(j * _L, _L)
            c16 = cid_v[sl]
            t16 = tt_v[sl]
            rowg = (base + j * _L) + i16
            new_ns = []
            for t in range(3):
                n_t = ns[t]
                m = t16 == (t + 1)
                mi = jnp.where(m, jnp.full((_L,), 1, jnp.int32),
                               jnp.zeros((_L,), jnp.int32))
                cs = plsc.cumsum(mi)
                dst = n_t + cs - jnp.full((_L,), 1, jnp.int32)
                dhi = jnp.right_shift(dst, jnp.full((_L,), 7, jnp.int32))
                dlo = jnp.bitwise_and(dst, jnp.full((_L,), 127, jnp.int32))
                plsc.store_scatter(gl[t], [dhi, dlo], c16, mask=m)
                plsc.store_scatter(sl_[t], [dhi, dlo], rowg, mask=m)
                new_ns.append(n_t + plsc.all_reduce_population_count(m))
            return tuple(new_ns)

        zeros = jnp.zeros((_L,), jnp.int32)
        ns = (zeros, zeros, zeros)
        for j in range(_PER_W // _L):
            ns = cvec(j, ns)
        counts = [jnp.max(ns[t]) for t in range(3)]
        tabs = [proc_hbm, med_hbm, chart_hbm]

        def g_copy(i):
            t, k = i // _NCHUNK, i % _NCHUNK
            return pltpu.make_async_copy(
                tabs[t].at[gl[t].at[k]], bufs_v.at[i % _NBUF], gsem)

        def s_copy(i):
            t, k = i // _NCHUNK, i % _NCHUNK
            return pltpu.make_async_copy(
                bufs_v.at[i % _NBUF], out_hbm.at[sl_[t].at[k]], ssem)

        def active(i):
            t, k = i // _NCHUNK, i % _NCHUNK
            return k * _CH < counts[t]

        NS = 3 * _NCHUNK
        for i in range(NS + _NBUF):
            if 0 <= i - _NBUF < NS:
                pl.when(active(i - _NBUF))(lambda ii=i - _NBUF: s_copy(ii).wait())
            if i < NS:
                pl.when(active(i))(lambda ii=i: g_copy(ii).start())
            if 0 <= i - 1 < NS:
                def _fire(ii=i - 1):
                    g_copy(ii).wait()
                    s_copy(ii).start()
                pl.when(active(i - 1))(_fire)

    return _sc_concept


_TB = 2048
_NB = N // _TB
_PI_2 = 1.5707963267948966


def _tc_body(cols_ref, cemb_ref, sel_ref, kadj_ref, stab_ref, tabs_ref,
             pW_ref, pb_ref, tw_ref, tb_ref,
             vW1_ref, vb1_ref, vW2_ref, vb2_ref, out_ref):
    f32 = jnp.float32
    P = jnp.dot(cols_ref[...], sel_ref[...])
    hot = (P[:, 0:256] == kadj_ref[...]).astype(f32)
    small_e = jnp.dot(hot, stab_ref[...])

    lane = lax.broadcasted_iota(jnp.int32, (1, H), 1).astype(f32)

    A = jnp.dot((P[:, 256:384] == lane).astype(f32), tabs_ref[:, 0:2 * H])
    Bc = jnp.dot((P[:, 384:512] == lane).astype(f32), tabs_ref[:, 2 * H:4 * H])
    pos_e = A[:, 0:H] * Bc[:, 0:H] + A[:, H:2 * H] * Bc[:, H:2 * H]

    TA = jnp.dot((P[:, 512:640] == lane).astype(f32), tabs_ref[:, 4 * H:6 * H])
    TB = jnp.dot((P[:, 640:768] == lane).astype(f32), tabs_ref[:, 6 * H:8 * H])
    per = TA[:, 0:H] * TB[:, 0:H] + TA[:, H:2 * H] * TB[:, H:2 * H]

    tmb = P[:, 768:896]
    u_row = jnp.dot(tw_ref[...], pW_ref[0:H, :])
    c_row = jnp.dot(tb_ref[...], pW_ref[0:H, :]) + pb_ref[...]
    time_e = tmb * u_row + c_row + jnp.dot(per, pW_ref[H:2 * H, :])

    vb = P[:, 896:1024]
    h1 = jnp.maximum(vb * vW1_ref[...] + vb1_ref[...], 0.0)
    val_e = jnp.dot(h1, vW2_ref[...]) + vb2_ref[...]

    ttb = P[:, 1024:1152]
    cemb = jnp.where(ttb >= 1.0, cemb_ref[...], jnp.zeros((), f32))

    out_ref[...] = cemb + pos_e + small_e + time_e + val_e


def _full_spec(r, c):
    return pl.BlockSpec((r, c), lambda i: (0, 0))


_tc_call = pl.pallas_call(
    _tc_body,
    grid=(_NB,),
    in_specs=[
        pl.BlockSpec((_TB, 16), lambda i: (i, 0)),
        pl.BlockSpec((_TB, H), lambda i: (i, 0)),
        _full_spec(16, 1152),
        _full_spec(1, 256),
        _full_spec(256, H),
        _full_spec(H, 8 * H),
        _full_spec(2 * H, H),
        _full_spec(1, H),
        _full_spec(1, H),
        _full_spec(1, H),
        _full_spec(1, H),
        _full_spec(1, H),
        _full_spec(H, H),
        _full_spec(1, H),
    ],
    out_specs=pl.BlockSpec((_TB, H), lambda i: (i, 0)),
    out_shape=jax.ShapeDtypeStruct((N, H), jnp.float32),
)


def _selector_constants():
    k = jnp.arange(1152)
    sel = jnp.zeros((16, 1152), jnp.float32)
    sel = sel.at[0].set(jnp.where(k < 128, 1.0, 0.0))
    sel = sel.at[1].set(jnp.where((k >= 128) & (k < 192), 1.0, 0.0))
    sel = sel.at[2].set(jnp.where((k >= 192) & (k < 195), 1.0, 0.0))
    sel = sel.at[3].set(jnp.where((k >= 195) & (k < 203), 1.0, 0.0))
    for c in range(7):
        sel = sel.at[4 + c].set(
            jnp.where((k >= 256 + 128 * c) & (k < 384 + 128 * c), 1.0, 0.0))
    k256 = jnp.arange(256)
    kadj = jnp.where(k256 < 128, k256.astype(jnp.float32), -1.0)
    kadj = jnp.where((k256 >= 128) & (k256 < 192), (k256 - 128).astype(jnp.float32), kadj)
    kadj = jnp.where((k256 >= 192) & (k256 < 195), (k256 - 192).astype(jnp.float32), kadj)
    kadj = jnp.where((k256 >= 195) & (k256 < 203), (k256 - 195).astype(jnp.float32), kadj)
    return sel, kadj.reshape(1, 256)


def _angle_tables(tfreqs, tb):
    f32 = jnp.float32
    i = jnp.arange(H, dtype=f32)
    w = jnp.power(10000.0, -2.0 * i / H)[None, :]
    ph = jnp.where(jnp.arange(H) % 2 == 0, 0.0, _PI_2)[None, :]
    n = jnp.arange(H, dtype=f32)[:, None]
    arg_a = 64.0 * n * w + ph
    arg_b = n * w
    f = tfreqs[None, :]
    arg_ta = (n / 64.0) * f + tb[None, :]
    arg_tb = (n / 4096.0) * f
    return jnp.concatenate(
        [jnp.sin(arg_a), jnp.cos(arg_a), jnp.cos(arg_b), jnp.sin(arg_b),
         jnp.sin(arg_ta), jnp.cos(arg_ta), jnp.cos(arg_tb), jnp.sin(arg_tb)],
        axis=1)


def kernel(concept, token_type, age, position, time, value, unit, gender, task,
           proc_table, med_table, chart_table, age_table, unit_table,
           gender_table, task_table, tw, tb, tfreqs, proj_W, proj_b,
           vW1, vb1, vW2, vb2):
    cemb = _build_sc_concept()(concept.reshape(N).astype(jnp.int32),
                               token_type.reshape(N).astype(jnp.int32),
                               proc_table, med_table, chart_table)

    stab = jnp.zeros((256, H), jnp.float32)
    stab = (stab.at[0:120].set(age_table)
                .at[128:192].set(unit_table)
                .at[192:195].set(gender_table)
                .at[195:203].set(task_table))

    f32 = jnp.float32
    tq = jnp.minimum(jnp.floor(time * 4096.0), 4095.0)
    t_hi = jnp.floor(tq * (1.0 / 64.0))
    t_lo = tq - 64.0 * t_hi
    z = jnp.zeros_like(time)
    cols = jnp.stack(
        [age.astype(f32), unit.astype(f32), gender.astype(f32),
         task.astype(f32), (position // 64).astype(f32),
         (position % 64).astype(f32), t_hi, t_lo, time, value,
         token_type.astype(f32), z, z, z, z, z], axis=-1).reshape(N, 16)
    sel, kadj = _selector_constants()
    tabs = _angle_tables(tfreqs, tb)

    out = _tc_call(cols, cemb, sel, kadj, stab, tabs, proj_W,
                   proj_b.reshape(1, H), tw, tb.reshape(1, H),
                   vW1, vb1.reshape(1, H), vW2, vb2.reshape(1, H))
    return out.reshape(B, S, H)


# speedup vs baseline: 13.6268x; 1.0039x over previous
import functools

import jax
import jax.numpy as jnp
from jax import lax
from jax.experimental import pallas as pl
from jax.experimental.pallas import tpu as pltpu
from jax.experimental.pallas import tpu_sc as plsc

B, S, H = 16, 2048, 128
N = B * S

_NC, _NS, _L = 2, 16, 16
_NW = _NC * _NS
_PER_W = N // _NW
_CH = 128
_NCHUNK = _PER_W // _CH

_TRASH = _NW * 3 * _CH
_NBUF = 6


@functools.cache
def _build_sc_concept():
    mesh = plsc.VectorSubcoreMesh(core_axis_name="c", subcore_axis_name="s")

    @functools.partial(
        pl.kernel,
        mesh=mesh,
        out_type=jax.ShapeDtypeStruct((N + _TRASH, H), jnp.float32),
        scratch_types=[
            pltpu.VMEM((_PER_W,), jnp.int32),
            pltpu.VMEM((_PER_W,), jnp.int32),
            pltpu.VMEM((_NCHUNK, _CH), jnp.int32),
            pltpu.VMEM((_NCHUNK, _CH), jnp.int32),
            pltpu.VMEM((_NCHUNK, _CH), jnp.int32),
            pltpu.VMEM((_NCHUNK, _CH), jnp.int32),
            pltpu.VMEM((_NCHUNK, _CH), jnp.int32),
            pltpu.VMEM((_NCHUNK, _CH), jnp.int32),
            pltpu.VMEM((_NBUF, _CH, H), jnp.float32),
            pltpu.SemaphoreType.DMA,
            pltpu.SemaphoreType.DMA,
        ],
        compiler_params=pltpu.CompilerParams(needs_layout_passes=False),
    )
    def _sc_concept(concept_hbm, tt_hbm, proc_hbm, med_hbm, chart_hbm,
                    out_hbm, cid_v, tt_v, g1_v, g2_v, g3_v, s1_v, s2_v, s3_v,
                    bufs_v, gsem, ssem):
        wid = lax.axis_index("s") * _NC + lax.axis_index("c")
        base = wid * _PER_W
        pltpu.sync_copy(concept_hbm.at[pl.ds(base, _PER_W)], cid_v)
        pltpu.sync_copy(tt_hbm.at[pl.ds(base, _PER_W)], tt_v)
        i16 = jnp.arange(16, dtype=jnp.int32)

        gl = [g1_v, g2_v, g3_v]
        sl_ = [s1_v, s2_v, s3_v]
        for t in range(3):
            tbase = N + (wid * 3 + t) * _CH
            for c in range(_CH // _L):
                fill_g = i16 + (c * _L)
                fill_s = i16 + (tbase + c * _L)
                for r in range(_NCHUNK):
                    gl[t][r, pl.ds(c * _L, _L)] = fill_g
                    sl_[t][r, pl.ds(c * _L, _L)] = fill_s

        def cvec(j, ns):
            sl = pl.ds(j * _L, _L)
            c16 = cid_v[sl]
            t16 = tt_v[sl]
            rowg = (base + j * _L) + i16
            new_ns = []
            for t in range(3):
                n_t = ns[t]
                m = t16 == (t + 1)
                mi = jnp.where(m, jnp.full((_L,), 1, jnp.int32),
                               jnp.zeros((_L,), jnp.int32))
                cs = plsc.cumsum(mi)
                dst = n_t + cs - jnp.full((_L,), 1, jnp.int32)
                dhi = jnp.right_shift(dst, jnp.full((_L,), 7, jnp.int32))
                dlo = jnp.bitwise_and(dst, jnp.full((_L,), 127, jnp.int32))
                plsc.store_scatter(gl[t], [dhi, dlo], c16, mask=m)
                plsc.store_scatter(sl_[t], [dhi, dlo], rowg, mask=m)
                new_ns.append(n_t + plsc.all_reduce_population_count(m))
            return tuple(new_ns)

        zeros = jnp.zeros((_L,), jnp.int32)
        ns = (zeros, zeros, zeros)
        for j in range(_PER_W // _L):
            ns = cvec(j, ns)
        counts = [jnp.max(ns[t]) for t in range(3)]
        tabs = [proc_hbm, med_hbm, chart_hbm]

        def g_copy(i):
            t, k = i // _NCHUNK, i % _NCHUNK
            return pltpu.make_async_copy(
                tabs[t].at[gl[t].at[k]], bufs_v.at[i % _NBUF], gsem)

        def s_copy(i):
            t, k = i // _NCHUNK, i % _NCHUNK
            return pltpu.make_async_copy(
                bufs_v.at[i % _NBUF], out_hbm.at[sl_[t].at[k]], ssem)

        def active(i):
            t, k = i // _NCHUNK, i % _NCHUNK
            return k * _CH < counts[t]

        NS = 3 * _NCHUNK
        for i in range(NS + _NBUF):
            if 0 <= i - _NBUF < NS:
                pl.when(active(i - _NBUF))(lambda ii=i - _NBUF: s_copy(ii).wait())
            if i < NS:
                pl.when(active(i))(lambda ii=i: g_copy(ii).start())
            if 0 <= i - 2 < NS:
                def _fire(ii=i - 2):
                    g_copy(ii).wait()
                    s_copy(ii).start()
                pl.when(active(i - 2))(_fire)

    return _sc_concept


_TB = 2048
_NB = N // _TB
_PI_2 = 1.5707963267948966


def _tc_body(cols_ref, cemb_ref, sel_ref, kadj_ref, stab_ref, tabs_ref,
             pW_ref, pb_ref, tw_ref, tb_ref,
             vW1_ref, vb1_ref, vW2_ref, vb2_ref, out_ref):
    f32 = jnp.float32
    P = jnp.dot(cols_ref[...], sel_ref[...])
    hot = (P[:, 0:256] == kadj_ref[...]).astype(f32)
    small_e = jnp.dot(hot, stab_ref[...])

    lane = lax.broadcasted_iota(jnp.int32, (1, H), 1).astype(f32)

    A = jnp.dot((P[:, 256:384] == lane).astype(f32), tabs_ref[:, 0:2 * H])
    Bc = jnp.dot((P[:, 384:512] == lane).astype(f32), tabs_ref[:, 2 * H:4 * H])
    pos_e = A[:, 0:H] * Bc[:, 0:H] + A[:, H:2 * H] * Bc[:, H:2 * H]

    TA = jnp.dot((P[:, 512:640] == lane).astype(f32), tabs_ref[:, 4 * H:6 * H])
    TB = jnp.dot((P[:, 640:768] == lane).astype(f32), tabs_ref[:, 6 * H:8 * H])
    per = TA[:, 0:H] * TB[:, 0:H] + TA[:, H:2 * H] * TB[:, H:2 * H]

    tmb = P[:, 768:896]
    u_row = jnp.dot(tw_ref[...], pW_ref[0:H, :])
    c_row = jnp.dot(tb_ref[...], pW_ref[0:H, :]) + pb_ref[...]
    time_e = tmb * u_row + c_row + jnp.dot(per, pW_ref[H:2 * H, :])

    vb = P[:, 896:1024]
    h1 = jnp.maximum(vb * vW1_ref[...] + vb1_ref[...], 0.0)
    val_e = jnp.dot(h1, vW2_ref[...]) + vb2_ref[...]

    ttb = P[:, 1024:1152]
    cemb = jnp.where(ttb >= 1.0, cemb_ref[...], jnp.zeros((), f32))

    out_ref[...] = cemb + pos_e + small_e + time_e + val_e


def _full_spec(r, c):
    return pl.BlockSpec((r, c), lambda i: (0, 0))


_tc_call = pl.pallas_call(
    _tc_body,
    grid=(_NB,),
    in_specs=[
        pl.BlockSpec((_TB, 16), lambda i: (i, 0)),
        pl.BlockSpec((_TB, H), lambda i: (i, 0)),
        _full_spec(16, 1152),
        _full_spec(1, 256),
        _full_spec(256, H),
        _full_spec(H, 8 * H),
        _full_spec(2 * H, H),
        _full_spec(1, H),
        _full_spec(1, H),
        _full_spec(1, H),
        _full_spec(1, H),
        _full_spec(1, H),
        _full_spec(H, H),
        _full_spec(1, H),
    ],
    out_specs=pl.BlockSpec((_TB, H), lambda i: (i, 0)),
    out_shape=jax.ShapeDtypeStruct((N, H), jnp.float32),
)


def _selector_constants():
    k = jnp.arange(1152)
    sel = jnp.zeros((16, 1152), jnp.float32)
    sel = sel.at[0].set(jnp.where(k < 128, 1.0, 0.0))
    sel = sel.at[1].set(jnp.where((k >= 128) & (k < 192), 1.0, 0.0))
    sel = sel.at[2].set(jnp.where((k >= 192) & (k < 195), 1.0, 0.0))
    sel = sel.at[3].set(jnp.where((k >= 195) & (k < 203), 1.0, 0.0))
    for c in range(7):
        sel = sel.at[4 + c].set(
            jnp.where((k >= 256 + 128 * c) & (k < 384 + 128 * c), 1.0, 0.0))
    k256 = jnp.arange(256)
    kadj = jnp.where(k256 < 128, k256.astype(jnp.float32), -1.0)
    kadj = jnp.where((k256 >= 128) & (k256 < 192), (k256 - 128).astype(jnp.float32), kadj)
    kadj = jnp.where((k256 >= 192) & (k256 < 195), (k256 - 192).astype(jnp.float32), kadj)
    kadj = jnp.where((k256 >= 195) & (k256 < 203), (k256 - 195).astype(jnp.float32), kadj)
    return sel, kadj.reshape(1, 256)


def _angle_tables(tfreqs, tb):
    f32 = jnp.float32
    i = jnp.arange(H, dtype=f32)
    w = jnp.power(10000.0, -2.0 * i / H)[None, :]
    ph = jnp.where(jnp.arange(H) % 2 == 0, 0.0, _PI_2)[None, :]
    n = jnp.arange(H, dtype=f32)[:, None]
    arg_a = 64.0 * n * w + ph
    arg_b = n * w
    f = tfreqs[None, :]
    arg_ta = (n / 64.0) * f + tb[None, :]
    arg_tb = (n / 4096.0) * f
    return jnp.concatenate(
        [jnp.sin(arg_a), jnp.cos(arg_a), jnp.cos(arg_b), jnp.sin(arg_b),
         jnp.sin(arg_ta), jnp.cos(arg_ta), jnp.cos(arg_tb), jnp.sin(arg_tb)],
        axis=1)


def kernel(concept, token_type, age, position, time, value, unit, gender, task,
           proc_table, med_table, chart_table, age_table, unit_table,
           gender_table, task_table, tw, tb, tfreqs, proj_W, proj_b,
           vW1, vb1, vW2, vb2):
    cemb = _build_sc_concept()(concept.reshape(N).astype(jnp.int32),
                               token_type.reshape(N).astype(jnp.int32),
                               proc_table, med_table, chart_table)

    stab = jnp.zeros((256, H), jnp.float32)
    stab = (stab.at[0:120].set(age_table)
                .at[128:192].set(unit_table)
                .at[192:195].set(gender_table)
                .at[195:203].set(task_table))

    f32 = jnp.float32
    tq = jnp.minimum(jnp.floor(time * 4096.0), 4095.0)
    t_hi = jnp.floor(tq * (1.0 / 64.0))
    t_lo = tq - 64.0 * t_hi
    z = jnp.zeros_like(time)
    cols = jnp.stack(
        [age.astype(f32), unit.astype(f32), gender.astype(f32),
         task.astype(f32), (position // 64).astype(f32),
         (position % 64).astype(f32), t_hi, t_lo, time, value,
         token_type.astype(f32), z, z, z, z, z], axis=-1).reshape(N, 16)
    sel, kadj = _selector_constants()
    tabs = _angle_tables(tfreqs, tb)

    out = _tc_call(cols, cemb, sel, kadj, stab, tabs, proj_W,
                   proj_b.reshape(1, H), tw, tb.reshape(1, H),
                   vW1, vb1.reshape(1, H), vW2, vb2.reshape(1, H))
    return out.reshape(B, S, H)


# speedup vs baseline: 13.6619x; 1.0026x over previous
import functools

import numpy as np

import jax
import jax.numpy as jnp
from jax import lax
from jax.experimental import pallas as pl
from jax.experimental.pallas import tpu as pltpu
from jax.experimental.pallas import tpu_sc as plsc

B, S, H = 16, 2048, 128
N = B * S

_NC, _NS, _L = 2, 16, 16
_NW = _NC * _NS
_PER_W = N // _NW
_CH = 128
_NCHUNK = _PER_W // _CH

_TRASH = _NW * 3 * _CH
_NBUF = 6


@functools.cache
def _build_sc_concept():
    mesh = plsc.VectorSubcoreMesh(core_axis_name="c", subcore_axis_name="s")

    @functools.partial(
        pl.kernel,
        mesh=mesh,
        out_type=jax.ShapeDtypeStruct((N + _TRASH, H), jnp.float32),
        scratch_types=[
            pltpu.VMEM((_PER_W,), jnp.int32),
            pltpu.VMEM((_PER_W,), jnp.int32),
            pltpu.VMEM((_NCHUNK, _CH), jnp.int32),
            pltpu.VMEM((_NCHUNK, _CH), jnp.int32),
            pltpu.VMEM((_NCHUNK, _CH), jnp.int32),
            pltpu.VMEM((_NCHUNK, _CH), jnp.int32),
            pltpu.VMEM((_NCHUNK, _CH), jnp.int32),
            pltpu.VMEM((_NCHUNK, _CH), jnp.int32),
            pltpu.VMEM((_NBUF, _CH, H), jnp.float32),
            pltpu.SemaphoreType.DMA,
            pltpu.SemaphoreType.DMA,
        ],
        compiler_params=pltpu.CompilerParams(needs_layout_passes=False),
    )
    def _sc_concept(concept_hbm, tt_hbm, proc_hbm, med_hbm, chart_hbm,
                    out_hbm, cid_v, tt_v, g1_v, g2_v, g3_v, s1_v, s2_v, s3_v,
                    bufs_v, gsem, ssem):
        wid = lax.axis_index("s") * _NC + lax.axis_index("c")
        base = wid * _PER_W
        pltpu.sync_copy(concept_hbm.at[pl.ds(base, _PER_W)], cid_v)
        pltpu.sync_copy(tt_hbm.at[pl.ds(base, _PER_W)], tt_v)
        i16 = jnp.arange(16, dtype=jnp.int32)

        gl = [g1_v, g2_v, g3_v]
        sl_ = [s1_v, s2_v, s3_v]
        for t in range(3):
            tbase = N + (wid * 3 + t) * _CH
            for c in range(_CH // _L):
                fill_g = i16 + (c * _L)
                fill_s = i16 + (tbase + c * _L)
                for r in range(_NCHUNK):
                    gl[t][r, pl.ds(c * _L, _L)] = fill_g
                    sl_[t][r, pl.ds(c * _L, _L)] = fill_s

        def cvec(j, ns):
            sl = pl.ds(j * _L, _L)
            c16 = cid_v[sl]
            t16 = tt_v[sl]
            rowg = (base + j * _L) + i16
            new_ns = []
            for t in range(3):
                n_t = ns[t]
                m = t16 == (t + 1)
                mi = jnp.where(m, jnp.full((_L,), 1, jnp.int32),
                               jnp.zeros((_L,), jnp.int32))
                cs = plsc.cumsum(mi)
                dst = n_t + cs - jnp.full((_L,), 1, jnp.int32)
                dhi = jnp.right_shift(dst, jnp.full((_L,), 7, jnp.int32))
                dlo = jnp.bitwise_and(dst, jnp.full((_L,), 127, jnp.int32))
                plsc.store_scatter(gl[t], [dhi, dlo], c16, mask=m)
                plsc.store_scatter(sl_[t], [dhi, dlo], rowg, mask=m)
                new_ns.append(n_t + plsc.all_reduce_population_count(m))
            return tuple(new_ns)

        zeros = jnp.zeros((_L,), jnp.int32)
        ns = (zeros, zeros, zeros)
        for j in range(_PER_W // _L):
            ns = cvec(j, ns)
        counts = [jnp.max(ns[t]) for t in range(3)]
        tabs = [proc_hbm, med_hbm, chart_hbm]

        def g_copy(i):
            t, k = i // _NCHUNK, i % _NCHUNK
            return pltpu.make_async_copy(
                tabs[t].at[gl[t].at[k]], bufs_v.at[i % _NBUF], gsem)

        def s_copy(i):
            t, k = i // _NCHUNK, i % _NCHUNK
            return pltpu.make_async_copy(
                bufs_v.at[i % _NBUF], out_hbm.at[sl_[t].at[k]], ssem)

        def active(i):
            t, k = i // _NCHUNK, i % _NCHUNK
            return k * _CH < counts[t]

        NS = 3 * _NCHUNK
        for i in range(NS + _NBUF):
            if 0 <= i - _NBUF < NS:
                pl.when(active(i - _NBUF))(lambda ii=i - _NBUF: s_copy(ii).wait())
            if i < NS:
                pl.when(active(i))(lambda ii=i: g_copy(ii).start())
            if 0 <= i - 2 < NS:
                def _fire(ii=i - 2):
                    g_copy(ii).wait()
                    s_copy(ii).start()
                pl.when(active(i - 2))(_fire)

    return _sc_concept


_TB = 2048
_NB = N // _TB
_PI_2 = 1.5707963267948966


def _tc_body(cols_ref, cemb_ref, sel_ref, kadj_ref, stab_ref, tabs_ref,
             pW_ref, pb_ref, tw_ref, tb_ref,
             vW1_ref, vb1_ref, vW2_ref, vb2_ref, out_ref):
    f32 = jnp.float32
    P = jnp.dot(cols_ref[...], sel_ref[...])
    hot = (P[:, 0:256] == kadj_ref[...]).astype(f32)
    small_e = jnp.dot(hot, stab_ref[...])

    lane = lax.broadcasted_iota(jnp.int32, (1, H), 1).astype(f32)

    A = jnp.dot((P[:, 256:384] == lane).astype(f32), tabs_ref[:, 0:2 * H])
    Bc = jnp.dot((P[:, 384:512] == lane).astype(f32), tabs_ref[:, 2 * H:4 * H])
    pos_e = A[:, 0:H] * Bc[:, 0:H] + A[:, H:2 * H] * Bc[:, H:2 * H]

    TA = jnp.dot((P[:, 512:640] == lane).astype(f32), tabs_ref[:, 4 * H:6 * H])
    TB = jnp.dot((P[:, 640:768] == lane).astype(f32), tabs_ref[:, 6 * H:8 * H])
    per = TA[:, 0:H] * TB[:, 0:H] + TA[:, H:2 * H] * TB[:, H:2 * H]

    tmb = P[:, 768:896]
    u_row = jnp.dot(tw_ref[...], pW_ref[0:H, :])
    c_row = jnp.dot(tb_ref[...], pW_ref[0:H, :]) + pb_ref[...]
    time_e = tmb * u_row + c_row + jnp.dot(per, pW_ref[H:2 * H, :])

    vb = P[:, 896:1024]
    h1 = jnp.maximum(vb * vW1_ref[...] + vb1_ref[...], 0.0)
    val_e = jnp.dot(h1, vW2_ref[...]) + vb2_ref[...]

    ttb = P[:, 1024:1152]
    cemb = jnp.where(ttb >= 1.0, cemb_ref[...], jnp.zeros((), f32))

    out_ref[...] = cemb + pos_e + small_e + time_e + val_e


def _full_spec(r, c):
    return pl.BlockSpec((r, c), lambda i: (0, 0))


_tc_call = pl.pallas_call(
    _tc_body,
    grid=(_NB,),
    in_specs=[
        pl.BlockSpec((_TB, 16), lambda i: (i, 0)),
        pl.BlockSpec((_TB, H), lambda i: (i, 0)),
        _full_spec(16, 1152),
        _full_spec(1, 256),
        _full_spec(256, H),
        _full_spec(H, 8 * H),
        _full_spec(2 * H, H),
        _full_spec(1, H),
        _full_spec(1, H),
        _full_spec(1, H),
        _full_spec(1, H),
        _full_spec(1, H),
        _full_spec(H, H),
        _full_spec(1, H),
    ],
    out_specs=pl.BlockSpec((_TB, H), lambda i: (i, 0)),
    out_shape=jax.ShapeDtypeStruct((N, H), jnp.float32),
)


def _selector_constants():
    k = np.arange(1152)
    sel = np.zeros((16, 1152), np.float32)
    sel[0] = (k < 128)
    sel[1] = (k >= 128) & (k < 192)
    sel[2] = (k >= 192) & (k < 195)
    sel[3] = (k >= 195) & (k < 203)
    for c in range(7):
        sel[4 + c] = (k >= 256 + 128 * c) & (k < 384 + 128 * c)
    k256 = np.arange(256)
    kadj = np.where(k256 < 128, k256.astype(np.float32), -1.0)
    kadj = np.where((k256 >= 128) & (k256 < 192), k256 - 128, kadj)
    kadj = np.where((k256 >= 192) & (k256 < 195), k256 - 192, kadj)
    kadj = np.where((k256 >= 195) & (k256 < 203), k256 - 195, kadj)
    return sel, kadj.reshape(1, 256).astype(np.float32)


_SEL_NP, _KADJ_NP = _selector_constants()


def _pos_tables_np():
    i = np.arange(H, dtype=np.float64)
    w = np.power(10000.0, -2.0 * i / H)[None, :]
    ph = np.where(np.arange(H) % 2 == 0, 0.0, _PI_2)[None, :]
    n = np.arange(H, dtype=np.float64)[:, None]
    arg_a = 64.0 * n * w + ph
    arg_b = n * w
    return np.concatenate(
        [np.sin(arg_a), np.cos(arg_a), np.cos(arg_b), np.sin(arg_b)],
        axis=1).astype(np.float32)


_POS_TABS_NP = _pos_tables_np()


def _angle_tables(tfreqs, tb):
    n = jnp.arange(H, dtype=jnp.float32)[:, None]
    f = tfreqs[None, :]
    arg_ta = (n / 64.0) * f + tb[None, :]
    arg_tb = (n / 4096.0) * f
    return jnp.concatenate(
        [jnp.asarray(_POS_TABS_NP),
         jnp.sin(arg_ta), jnp.cos(arg_ta), jnp.cos(arg_tb), jnp.sin(arg_tb)],
        axis=1)


def kernel(concept, token_type, age, position, time, value, unit, gender, task,
           proc_table, med_table, chart_table, age_table, unit_table,
           gender_table, task_table, tw, tb, tfreqs, proj_W, proj_b,
           vW1, vb1, vW2, vb2):
    cemb = _build_sc_concept()(concept.reshape(N).astype(jnp.int32),
                               token_type.reshape(N).astype(jnp.int32),
                               proc_table, med_table, chart_table)

    stab = jnp.zeros((256, H), jnp.float32)
    stab = (stab.at[0:120].set(age_table)
                .at[128:192].set(unit_table)
                .at[192:195].set(gender_table)
                .at[195:203].set(task_table))

    f32 = jnp.float32
    tq = jnp.minimum(jnp.floor(time * 4096.0), 4095.0)
    t_hi = jnp.floor(tq * (1.0 / 64.0))
    t_lo = tq - 64.0 * t_hi
    z = jnp.zeros_like(time)
    cols = jnp.stack(
        [age.astype(f32), unit.astype(f32), gender.astype(f32),
         task.astype(f32), (position // 64).astype(f32),
         (position % 64).astype(f32), t_hi, t_lo, time, value,
         token_type.astype(f32), z, z, z, z, z], axis=-1).reshape(N, 16)
    sel = jnp.asarray(_SEL_NP)
    kadj = jnp.asarray(_KADJ_NP)
    tabs = _angle_tables(tfreqs, tb)

    out = _tc_call(cols, cemb, sel, kadj, stab, tabs, proj_W,
                   proj_b.reshape(1, H), tw, tb.reshape(1, H),
                   vW1, vb1.reshape(1, H), vW2, vb2.reshape(1, H))
    return out.reshape(B, S, H)
